# baseline (speedup 1.0000x reference)
import jax
import jax.numpy as jnp
from jax import lax
from jax.experimental import pallas as pl
from jax.experimental.pallas import tpu as pltpu
from jax.experimental.pallas import tpu_sc as plsc

N_LEVELS = 16
N_FEATURES = 2
LOG2_HASH = 19
HASH_SIZE = 1 << LOG2_HASH
HMASK = HASH_SIZE - 1
BASE_RES = 16
B_PTS = 1048576

_RES = [BASE_RES << l for l in range(N_LEVELS)]

NC = 2
NS = 16
NW = NC * NS
LANES = 16

PT_PER_TILE = B_PTS // NW
ROW_W = 8
LOG2_ROW_W = 3
RSH = 5 - LOG2_ROW_W
CHUNK = 128
N_CHUNKS = PT_PER_TILE // CHUNK
X_STAGE = 2048
CHUNKS_PER_STAGE = X_STAGE // CHUNK
GCS = 128
GCM = 2 * CHUNK

N_SMALL = 11
BIG = list(range(N_SMALL, N_LEVELS))
N_BIG = len(BIG)

_R = [_RES[l] + 2 for l in range(N_SMALL)]
_OFF = []
_o = 0
for _l in range(N_SMALL):
    _OFF.append(_o)
    _o += -(-(2 * _R[_l]) // 8) * 8
TAB_WORDS = _o + 512

OUT_THIRD = CHUNK * 2 * N_LEVELS
HALF_I = N_BIG * 2 * CHUNK
ROW_CAP = 2 * HALF_I


def _body(x_hbm, embr, out_hbm, tab, xbuf, bigidx, rows, outb, sem_g, sem_o):
    wid = lax.axis_index("s") * NC + lax.axis_index("c")
    tile_base = wid * PT_PER_TILE

    iota = lax.iota(jnp.int32, LANES)
    lane32 = iota * (2 * N_LEVELS)
    half = iota >> 1
    parity = iota & 1

    for l in range(N_SMALL):
        rmax = _R[l] - 1
        for rs in range(0, _R[l], ROW_CAP):
            n = min(ROW_CAP, _R[l] - rs)
            n_pad = -(-n // GCS) * GCS

            def _stage_idx(k, _, l=l, rs=rs, rmax=rmax):
                j = jnp.minimum(iota + (rs + k * LANES), rmax)
                bigidx[pl.ds(k * LANES, LANES)] = (
                    (j << RSH) + ((2 * l) >> LOG2_ROW_W))
                return 0

            lax.fori_loop(0, n_pad // LANES, _stage_idx, 0)
            for cc in range(0, n_pad, GCS):
                pltpu.async_copy(
                    embr.at[bigidx.at[pl.ds(cc, GCS)]],
                    rows.at[pl.ds(cc, GCS)],
                    sem_g,
                )
            for cc in range(0, n_pad, GCS):
                pltpu.make_async_copy(
                    embr.at[bigidx.at[pl.ds(cc, GCS)]],
                    rows.at[pl.ds(cc, GCS)],
                    sem_g,
                ).wait()

            def _extract(m, _, l=l, rs=rs):
                v = plsc.load_gather(
                    rows, [half + m * 8, parity + ((2 * l) % ROW_W)])
                tab[pl.ds(_OFF[l] + 2 * rs + m * LANES, LANES)] = v
                return 0

            lax.fori_loop(0, (2 * n_pad) // LANES, _extract, 0)

    def xoff_of(g):
        return (lax.rem(g // CHUNKS_PER_STAGE, 2) * X_STAGE
                + lax.rem(g, CHUNKS_PER_STAGE) * CHUNK)

    def p1a(g, ho):
        xo = xoff_of(g)

        def it(k, _):
            x16 = xbuf[pl.ds(xo + k * LANES, LANES)]
            x16 = jnp.minimum(jnp.maximum(x16, 0.0), 1.0)
            for l in BIG:
                bl = l - N_SMALL
                pos = x16 * float(_RES[l])
                i0 = pos.astype(jnp.int32)
                h0 = i0
                h1 = i0 + 1
                if l == N_LEVELS - 1:
                    h0 = h0 & HMASK
                    h1 = h1 & HMASK
                rsub = (2 * l) >> LOG2_ROW_W
                bigidx[pl.ds(ho + bl * 2 * CHUNK + k * LANES, LANES)] = (
                    (h0 << RSH) + rsub)
                bigidx[pl.ds(ho + bl * 2 * CHUNK + CHUNK + k * LANES,
                             LANES)] = ((h1 << RSH) + rsub)
            return 0

        lax.fori_loop(0, CHUNK // LANES, it, 0)

    def fire(ho):
        for bl in range(N_BIG):
            pltpu.async_copy(
                embr.at[bigidx.at[pl.ds(ho + bl * GCM, GCM)]],
                rows.at[pl.ds(ho + bl * GCM, GCM)],
                sem_g,
            )

    def drain(ho):
        for bl in range(N_BIG):
            pltpu.make_async_copy(
                embr.at[bigidx.at[pl.ds(ho + bl * GCM, GCM)]],
                rows.at[pl.ds(ho + bl * GCM, GCM)],
                sem_g,
            ).wait()

    def p1b(g):
        xo = xoff_of(g)
        ob = lax.rem(g, 3) * OUT_THIRD

        def it(k, _):
            x16 = xbuf[pl.ds(xo + k * LANES, LANES)]
            x16 = jnp.minimum(jnp.maximum(x16, 0.0), 1.0)
            rb32 = lane32 + (ob + k * (LANES * 2 * N_LEVELS))
            for l in range(N_SMALL):
                pos = x16 * float(_RES[l])
                i0 = pos.astype(jnp.int32)
                w = pos - i0.astype(jnp.float32)
                a = (i0 << 1) + _OFF[l]
                e0x = plsc.load_gather(tab, [a])
                e0y = plsc.load_gather(tab, [a + 1])
                e1x = plsc.load_gather(tab, [a + 2])
                e1y = plsc.load_gather(tab, [a + 3])
                ox = e0x + w * (e1x - e0x)
                oy = e0y + w * (e1y - e0y)
                plsc.store_scatter(outb, [rb32 + 2 * l], ox)
                plsc.store_scatter(outb, [rb32 + (2 * l + 1)], oy)
            return 0

        lax.fori_loop(0, CHUNK // LANES, it, 0)

    def p2(g, ho):
        xo = xoff_of(g)
        ob = lax.rem(g, 3) * OUT_THIRD

        def it(m, _):
            x16 = xbuf[pl.ds(xo + m * LANES, LANES)]
            x16 = jnp.minimum(jnp.maximum(x16, 0.0), 1.0)
            rb32 = lane32 + (ob + m * (LANES * 2 * N_LEVELS))
            for l in BIG:
                bl = l - N_SMALL
                pos = x16 * float(_RES[l])
                i0 = pos.astype(jnp.int32)
                w = pos - i0.astype(jnp.float32)
                rA = iota + (ho + bl * 2 * CHUNK + m * LANES)
                rB = rA + CHUNK
                cx = iota * 0 + ((2 * l) % ROW_W)
                e0x = plsc.load_gather(rows, [rA, cx])
                e0y = plsc.load_gather(rows, [rA, cx + 1])
                e1x = plsc.load_gather(rows, [rB, cx])
                e1y = plsc.load_gather(rows, [rB, cx + 1])
                ox = e0x + w * (e1x - e0x)
                oy = e0y + w * (e1y - e0y)
                plsc.store_scatter(outb, [rb32 + 2 * l], ox)
                plsc.store_scatter(outb, [rb32 + (2 * l + 1)], oy)
            return 0

        lax.fori_loop(0, CHUNK // LANES, it, 0)

    def fire_out(g):
        pltpu.async_copy(
            outb.at[pl.ds(lax.rem(g, 3) * OUT_THIRD, OUT_THIRD)],
            out_hbm.at[pl.ds((tile_base + g * CHUNK) * 2 * N_LEVELS,
                             OUT_THIRD)],
            sem_o,
        )

    def drain_out(g):
        pltpu.make_async_copy(
            outb.at[pl.ds(lax.rem(g, 3) * OUT_THIRD, OUT_THIRD)],
            out_hbm.at[pl.ds((tile_base + g * CHUNK) * 2 * N_LEVELS,
                             OUT_THIRD)],
            sem_o,
        ).wait()

    def _iter(g2, _):
        ga = 2 * g2
        gb = ga + 1

        @pl.when(lax.rem(ga, CHUNKS_PER_STAGE) == 0)
        def _():
            st = ga // CHUNKS_PER_STAGE
            pltpu.sync_copy(
                x_hbm.at[pl.ds(tile_base + st * X_STAGE, X_STAGE)],
                xbuf.at[pl.ds(lax.rem(st, 2) * X_STAGE, X_STAGE)],
            )

        p1a(ga, 0)
        fire(0)

        @pl.when(g2 >= 1)
        def _():
            drain(HALF_I)
            p2(ga - 1, HALF_I)
            fire_out(ga - 1)

        @pl.when(ga >= 3)
        def _():
            drain_out(ga - 3)

        p1b(ga)

        p1a(gb, HALF_I)
        fire(HALF_I)

        drain(0)
        p2(ga, 0)
        fire_out(ga)

        @pl.when(gb >= 3)
        def _():
            drain_out(gb - 3)

        p1b(gb)
        return 0

    lax.fori_loop(0, N_CHUNKS // 2, _iter, 0)

    glast = N_CHUNKS - 1
    drain(HALF_I)
    p2(glast, HALF_I)
    fire_out(glast)
    for gg in (N_CHUNKS - 3, N_CHUNKS - 2, N_CHUNKS - 1):
        drain_out(gg)


@jax.jit
def kernel(x, embeddings):
    assert x.shape == (B_PTS,) and embeddings.shape == (HASH_SIZE,
                                                        2 * N_LEVELS)
    embr = embeddings.reshape(HASH_SIZE * 2 * N_LEVELS // ROW_W, ROW_W)
    mesh = plsc.VectorSubcoreMesh(core_axis_name="c", subcore_axis_name="s")
    out = pl.kernel(
        _body,
        out_type=jax.ShapeDtypeStruct((B_PTS * 2 * N_LEVELS,), jnp.float32),
        mesh=mesh,
        compiler_params=pltpu.CompilerParams(
            use_tc_tiling_on_sc=False, needs_layout_passes=False),
        scratch_types=[
            pltpu.VMEM((TAB_WORDS,), jnp.float32),
            pltpu.VMEM((2 * X_STAGE,), jnp.float32),
            pltpu.VMEM((2 * HALF_I,), jnp.int32),
            pltpu.VMEM((ROW_CAP, ROW_W), jnp.float32),
            pltpu.VMEM((3 * OUT_THIRD,), jnp.float32),
            pltpu.SemaphoreType.DMA,
            pltpu.SemaphoreType.DMA,
        ],
    )(x, embr)
    return out.reshape(B_PTS, 2 * N_LEVELS)
